# baseline (speedup 1.0000x reference)
import functools

import jax
import jax.numpy as jnp
from jax import lax
from jax.experimental import pallas as pl
from jax.experimental.pallas import tpu as pltpu
from jax.experimental.pallas import tpu_sc as plsc

B = 16384
D = 64
NUM_E = 1000000
NC = 2
NS = 16
NW = NC * NS
SEG = B // NW
QSEG = SEG // 4
CHW = 768
TAIL_C = NUM_E // CHW
TAIL_START = TAIL_C * CHW
TAIL_PAD = 128
ROW_PAD = 128

_mesh = plsc.VectorSubcoreMesh(core_axis_name="c", subcore_axis_name="s")
_params = pltpu.CompilerParams(
    use_tc_tiling_on_sc=True, needs_layout_passes=False
)


def _lane(vec, lane):
    sel = lax.broadcasted_iota(jnp.int32, (16,), 0) == lane
    return jnp.sum(jnp.where(sel, vec, 0))


@functools.partial(
    pl.kernel,
    mesh=_mesh,
    out_type=jax.ShapeDtypeStruct((B, ROW_PAD), jnp.float32),
    scratch_types=[
        pltpu.VMEM((SEG,), jnp.int32),
        pltpu.VMEM((2, 8, 8, CHW), jnp.float32),
        pltpu.VMEM((QSEG, ROW_PAD), jnp.float32),
        pltpu.SemaphoreType.DMA,
        pltpu.SemaphoreType.DMA,
        pltpu.SemaphoreType.DMA,
    ],
    compiler_params=_params,
)
def _gather_sigmoid(es_hbm, tbl_hbm, tail_hbm, res_hbm,
                    es_v, slab_v, out_v, sem0, sem1, semf):
    wid = lax.axis_index("s") * NC + lax.axis_index("c")
    seg0 = wid * SEG

    pltpu.sync_copy(es_hbm.at[pl.ds(seg0, SEG)], es_v)

    e_first = _lane(es_v[pl.ds(0, 16)], 0)
    e_last = _lane(es_v[pl.ds(SEG - 16, 16)], 15)
    c_lo = e_first // CHW
    c_hi = e_last // CHW
    cnt = c_hi - c_lo + 1

    sems = (sem0, sem1)

    def chunk_copy(k, p):
        c = c_lo + k
        return pltpu.make_async_copy(
            tbl_hbm.at[:, :, pl.ds(c * CHW, CHW)], slab_v.at[p], sems[p]
        )

    def tail_copy(p):
        return pltpu.make_async_copy(
            tail_hbm, slab_v.at[p, :, :, pl.ds(0, TAIL_PAD)], sems[p]
        )

    def start_chunk(k, p):
        c = c_lo + k

        @pl.when(c != TAIL_C)
        def _():
            chunk_copy(k, p).start()

        @pl.when(c == TAIL_C)
        def _():
            tail_copy(p).start()

    def wait_chunk(k, p):
        c = c_lo + k

        @pl.when(c != TAIL_C)
        def _():
            chunk_copy(k, p).wait()

        @pl.when(c == TAIL_C)
        def _():
            tail_copy(p).wait()

    lanes = lax.broadcasted_iota(jnp.int32, (16,), 0)

    def process_chunk(k, p, carry):
        c = c_lo + k
        eb = c * CHW
        hi = eb + jnp.where(c == TAIL_C, NUM_E - TAIL_START, CHW)

        def cond(st):
            _, _, done = st
            return jnp.logical_not(done)

        def body(st):
            g_, flushed, done = st

            quarter = jnp.minimum(g_ // (QSEG // 16), 3)

            @pl.when(quarter > flushed)
            def _():
                pltpu.async_copy(
                    out_v, res_hbm.at[pl.ds(seg0 + flushed * QSEG, QSEG)],
                    semf,
                ).wait()

            flushed_n = jnp.maximum(flushed, quarter)

            ev = es_v[pl.ds(g_ * 16, 16)]
            in_mask = (ev >= eb) & (ev < hi)
            el = jnp.where(in_mask, ev - eb, 0)
            rows = jnp.where(
                in_mask, (g_ % (QSEG // 16)) * 16 + lanes, 0
            )
            pv = jnp.full((16,), p, dtype=jnp.int32)
            for h in range(D):
                fr = jnp.full((16,), h // 8, dtype=jnp.int32)
                fc = jnp.full((16,), h % 8, dtype=jnp.int32)
                hv = jnp.full((16,), h, dtype=jnp.int32)
                v = plsc.load_gather(slab_v, [pv, fr, fc, el])
                v = 1.0 / (1.0 + jnp.exp(-v))
                plsc.store_scatter(out_v, [rows, hv], v, mask=in_mask)
            adv = jnp.max(ev) < hi
            g_n = jnp.where(adv, g_ + 1, g_)
            done_n = jnp.logical_not(adv) | (g_n >= SEG // 16)
            return (g_n, flushed_n, done_n)

        g, flushed, _ = lax.while_loop(
            cond, body, (carry[0], carry[1], jnp.bool_(False))
        )
        return (g, flushed)

    start_chunk(0, 0)

    def pair_body(j, carry):
        k0 = 2 * j
        k1 = 2 * j + 1

        @pl.when(k1 < cnt)
        def _():
            start_chunk(k1, 1)

        def do0(st):
            wait_chunk(k0, 0)
            return process_chunk(k0, 0, st)

        carry = lax.cond(k0 < cnt, do0, lambda st: st, carry)

        @pl.when(k1 + 1 < cnt)
        def _():
            start_chunk(k1 + 1, 0)

        def do1(st):
            wait_chunk(k1, 1)
            return process_chunk(k1, 1, st)

        carry = lax.cond(k1 < cnt, do1, lambda st: st, carry)
        return carry

    lax.fori_loop(
        0, (cnt + 1) // 2, pair_body, (jnp.int32(0), jnp.int32(0))
    )

    pltpu.async_copy(
        out_v, res_hbm.at[pl.ds(seg0 + 3 * QSEG, QSEG)], semf
    ).wait()


@functools.partial(
    pl.kernel,
    mesh=_mesh,
    out_type=jax.ShapeDtypeStruct((D, B), jnp.float32),
    scratch_types=[
        pltpu.VMEM((SEG,), jnp.int32),
        pltpu.VMEM((SEG, ROW_PAD), jnp.float32),
        pltpu.VMEM((D, SEG), jnp.float32),
        pltpu.SemaphoreType.DMA,
    ],
    compiler_params=_params,
)
def _unsort(inv_hbm, res_hbm, out_hbm, inv_v, rows_v, stage_t, sem):
    wid = lax.axis_index("s") * NC + lax.axis_index("c")
    base = wid * SEG
    pltpu.sync_copy(inv_hbm.at[pl.ds(base, SEG)], inv_v)
    copies = []
    for g in range(SEG // 16):
        iv = inv_v[pl.ds(g * 16, 16)]
        copies.append(
            pltpu.async_copy(
                res_hbm.at[iv], rows_v.at[pl.ds(g * 16, 16)], sem
            )
        )
    for c in copies:
        c.wait()

    lanes = lax.broadcasted_iota(jnp.int32, (16,), 0)

    def tr_body(g, _):
        rvec = g * 16 + lanes
        for h in range(D):
            hv = jnp.full((16,), h, dtype=jnp.int32)
            stage_t[h, pl.ds(g * 16, 16)] = plsc.load_gather(
                rows_v, [rvec, hv]
            )
        return 0

    lax.fori_loop(0, SEG // 16, tr_body, 0)
    pltpu.sync_copy(stage_t, out_hbm.at[:, pl.ds(base, SEG)])


def kernel(X, Y, gate_theta):
    del X
    y32 = Y.astype(jnp.int32)
    iota = lax.broadcasted_iota(jnp.int32, (B,), 0)
    es, order = lax.sort([y32, iota], num_keys=1)
    inv = jnp.zeros((B,), jnp.int32).at[order].set(iota)
    tbl = gate_theta.T.reshape(8, 8, NUM_E)
    tail = jnp.pad(
        tbl[:, :, TAIL_START:],
        ((0, 0), (0, 0), (0, TAIL_PAD - (NUM_E - TAIL_START))),
    )
    res = _gather_sigmoid(es, tbl, tail)
    out_t = _unsort(inv, res)
    return out_t.T


# speedup vs baseline: 1.0927x; 1.0927x over previous
import functools

import jax
import jax.numpy as jnp
from jax import lax
from jax.experimental import pallas as pl
from jax.experimental.pallas import tpu as pltpu
from jax.experimental.pallas import tpu_sc as plsc

B = 16384
D = 64
NUM_E = 1000000
NC = 2
NS = 16
NW = NC * NS
SEG = B // NW
QSEG = SEG // 4
CHW = 768
TAIL_C = NUM_E // CHW
TAIL_START = TAIL_C * CHW
TAIL_PAD = 128
ROW_PAD = 128

_mesh = plsc.VectorSubcoreMesh(core_axis_name="c", subcore_axis_name="s")
_params = pltpu.CompilerParams(
    use_tc_tiling_on_sc=True, needs_layout_passes=False
)


def _lane(vec, lane):
    sel = lax.broadcasted_iota(jnp.int32, (16,), 0) == lane
    return jnp.sum(jnp.where(sel, vec, 0))


@functools.partial(
    pl.kernel,
    mesh=_mesh,
    out_type=jax.ShapeDtypeStruct((B, ROW_PAD), jnp.float32),
    scratch_types=[
        pltpu.VMEM((SEG,), jnp.int32),
        pltpu.VMEM((2, 8, 8, CHW), jnp.float32),
        pltpu.VMEM((QSEG, ROW_PAD), jnp.float32),
        pltpu.SemaphoreType.DMA,
        pltpu.SemaphoreType.DMA,
        pltpu.SemaphoreType.DMA,
    ],
    compiler_params=_params,
)
def _gather_sigmoid(es_hbm, tbl_hbm, tail_hbm, res_hbm,
                    es_v, slab_v, out_v, sem0, sem1, semf):
    wid = lax.axis_index("s") * NC + lax.axis_index("c")
    seg0 = wid * SEG

    pltpu.sync_copy(es_hbm.at[pl.ds(seg0, SEG)], es_v)

    e_first = _lane(es_v[pl.ds(0, 16)], 0)
    e_last = _lane(es_v[pl.ds(SEG - 16, 16)], 15)
    c_lo = e_first // CHW
    c_hi = e_last // CHW
    cnt = c_hi - c_lo + 1

    sems = (sem0, sem1)

    def chunk_copy(k, p):
        c = c_lo + k
        return pltpu.make_async_copy(
            tbl_hbm.at[:, :, pl.ds(c * CHW, CHW)], slab_v.at[p], sems[p]
        )

    def tail_copy(p):
        return pltpu.make_async_copy(
            tail_hbm, slab_v.at[p, :, :, pl.ds(0, TAIL_PAD)], sems[p]
        )

    def start_chunk(k, p):
        c = c_lo + k

        @pl.when(c != TAIL_C)
        def _():
            chunk_copy(k, p).start()

        @pl.when(c == TAIL_C)
        def _():
            tail_copy(p).start()

    def wait_chunk(k, p):
        c = c_lo + k

        @pl.when(c != TAIL_C)
        def _():
            chunk_copy(k, p).wait()

        @pl.when(c == TAIL_C)
        def _():
            tail_copy(p).wait()

    lanes = lax.broadcasted_iota(jnp.int32, (16,), 0)

    def process_chunk(k, p, carry):
        c = c_lo + k
        eb = c * CHW
        hi = eb + jnp.where(c == TAIL_C, NUM_E - TAIL_START, CHW)

        def cond(st):
            _, _, done = st
            return jnp.logical_not(done)

        def body(st):
            g_, flushed, done = st

            quarter = jnp.minimum(g_ // (QSEG // 16), 3)

            @pl.when(quarter > flushed)
            def _():
                pltpu.async_copy(
                    out_v, res_hbm.at[pl.ds(seg0 + flushed * QSEG, QSEG)],
                    semf,
                ).wait()

            flushed_n = jnp.maximum(flushed, quarter)

            ev = es_v[pl.ds(g_ * 16, 16)]
            in_mask = (ev >= eb) & (ev < hi)
            el = jnp.where(in_mask, ev - eb, 0)
            rows = jnp.where(
                in_mask, (g_ % (QSEG // 16)) * 16 + lanes, 0
            )
            pv = jnp.full((16,), p, dtype=jnp.int32)
            for h in range(D):
                fr = jnp.full((16,), h // 8, dtype=jnp.int32)
                fc = jnp.full((16,), h % 8, dtype=jnp.int32)
                hv = jnp.full((16,), h, dtype=jnp.int32)
                v = plsc.load_gather(slab_v, [pv, fr, fc, el])
                v = 1.0 / (1.0 + jnp.exp(-v))
                plsc.store_scatter(out_v, [rows, hv], v, mask=in_mask)
            adv = jnp.max(ev) < hi
            g_n = jnp.where(adv, g_ + 1, g_)
            done_n = jnp.logical_not(adv) | (g_n >= SEG // 16)
            return (g_n, flushed_n, done_n)

        g, flushed, _ = lax.while_loop(
            cond, body, (carry[0], carry[1], jnp.bool_(False))
        )
        return (g, flushed)

    start_chunk(0, 0)

    def pair_body(j, carry):
        k0 = 2 * j
        k1 = 2 * j + 1

        @pl.when(k1 < cnt)
        def _():
            start_chunk(k1, 1)

        def do0(st):
            wait_chunk(k0, 0)
            return process_chunk(k0, 0, st)

        carry = lax.cond(k0 < cnt, do0, lambda st: st, carry)

        @pl.when(k1 + 1 < cnt)
        def _():
            start_chunk(k1 + 1, 0)

        def do1(st):
            wait_chunk(k1, 1)
            return process_chunk(k1, 1, st)

        carry = lax.cond(k1 < cnt, do1, lambda st: st, carry)
        return carry

    lax.fori_loop(
        0, (cnt + 1) // 2, pair_body, (jnp.int32(0), jnp.int32(0))
    )

    pltpu.async_copy(
        out_v, res_hbm.at[pl.ds(seg0 + 3 * QSEG, QSEG)], semf
    ).wait()


@functools.partial(
    pl.kernel,
    mesh=_mesh,
    out_type=jax.ShapeDtypeStruct((B, ROW_PAD), jnp.float32),
    scratch_types=[
        pltpu.VMEM((SEG,), jnp.int32),
        pltpu.VMEM((SEG, ROW_PAD), jnp.float32),
        pltpu.SemaphoreType.DMA,
    ],
    compiler_params=_params,
)
def _unsort(inv_hbm, res_hbm, out_hbm, inv_v, rows_v, sem):
    wid = lax.axis_index("s") * NC + lax.axis_index("c")
    base = wid * SEG
    pltpu.sync_copy(inv_hbm.at[pl.ds(base, SEG)], inv_v)
    copies = []
    for g in range(SEG // 16):
        iv = inv_v[pl.ds(g * 16, 16)]
        copies.append(
            pltpu.async_copy(
                res_hbm.at[iv], rows_v.at[pl.ds(g * 16, 16)], sem
            )
        )
    for c in copies:
        c.wait()
    pltpu.sync_copy(rows_v, out_hbm.at[pl.ds(base, SEG)])


def kernel(X, Y, gate_theta):
    del X
    y32 = Y.astype(jnp.int32)
    iota = lax.broadcasted_iota(jnp.int32, (B,), 0)
    es, order = lax.sort([y32, iota], num_keys=1)
    inv = jnp.zeros((B,), jnp.int32).at[order].set(iota)
    tbl = gate_theta.T.reshape(8, 8, NUM_E)
    tail = jnp.pad(
        tbl[:, :, TAIL_START:],
        ((0, 0), (0, 0), (0, TAIL_PAD - (NUM_E - TAIL_START))),
    )
    res = _gather_sigmoid(es, tbl, tail)
    out = _unsort(inv, res)
    return out[:, :D]


# speedup vs baseline: 1.1479x; 1.0506x over previous
import functools

import jax
import jax.numpy as jnp
from jax import lax
from jax.experimental import pallas as pl
from jax.experimental.pallas import tpu as pltpu
from jax.experimental.pallas import tpu_sc as plsc

B = 16384
D = 64
NUM_E = 1000000
NC = 2
NS = 16
NW = NC * NS
SEG = B // NW
QSEG = SEG // 8
NFLUSH = SEG // QSEG
CHW = 896
TAIL_C = NUM_E // CHW
TAIL_START = TAIL_C * CHW
TAIL_PAD = 128
ROW_PAD = 128

_mesh = plsc.VectorSubcoreMesh(core_axis_name="c", subcore_axis_name="s")
_params = pltpu.CompilerParams(
    use_tc_tiling_on_sc=True, needs_layout_passes=False
)


def _lane(vec, lane):
    sel = lax.broadcasted_iota(jnp.int32, (16,), 0) == lane
    return jnp.sum(jnp.where(sel, vec, 0))


@functools.partial(
    pl.kernel,
    mesh=_mesh,
    out_type=jax.ShapeDtypeStruct((B, ROW_PAD), jnp.float32),
    scratch_types=[
        pltpu.VMEM((SEG,), jnp.int32),
        pltpu.VMEM((2, 8, 8, CHW), jnp.float32),
        pltpu.VMEM((QSEG, ROW_PAD), jnp.float32),
        pltpu.SemaphoreType.DMA,
        pltpu.SemaphoreType.DMA,
        pltpu.SemaphoreType.DMA,
    ],
    compiler_params=_params,
)
def _gather_sigmoid(es_hbm, tbl_hbm, tail_hbm, res_hbm,
                    es_v, slab_v, out_v, sem0, sem1, semf):
    wid = lax.axis_index("s") * NC + lax.axis_index("c")
    seg0 = wid * SEG

    pltpu.sync_copy(es_hbm.at[pl.ds(seg0, SEG)], es_v)

    e_first = _lane(es_v[pl.ds(0, 16)], 0)
    e_last = _lane(es_v[pl.ds(SEG - 16, 16)], 15)
    c_lo = e_first // CHW
    c_hi = e_last // CHW
    cnt = c_hi - c_lo + 1

    sems = (sem0, sem1)

    def chunk_copy(k, p):
        c = c_lo + k
        return pltpu.make_async_copy(
            tbl_hbm.at[:, :, pl.ds(c * CHW, CHW)], slab_v.at[p], sems[p]
        )

    def tail_copy(p):
        return pltpu.make_async_copy(
            tail_hbm, slab_v.at[p, :, :, pl.ds(0, TAIL_PAD)], sems[p]
        )

    def start_chunk(k, p):
        c = c_lo + k

        @pl.when(c != TAIL_C)
        def _():
            chunk_copy(k, p).start()

        @pl.when(c == TAIL_C)
        def _():
            tail_copy(p).start()

    def wait_chunk(k, p):
        c = c_lo + k

        @pl.when(c != TAIL_C)
        def _():
            chunk_copy(k, p).wait()

        @pl.when(c == TAIL_C)
        def _():
            tail_copy(p).wait()

    lanes = lax.broadcasted_iota(jnp.int32, (16,), 0)

    def process_chunk(k, p, carry):
        c = c_lo + k
        eb = c * CHW
        hi = eb + jnp.where(c == TAIL_C, NUM_E - TAIL_START, CHW)

        def cond(st):
            _, _, done = st
            return jnp.logical_not(done)

        def body(st):
            g_, flushed, done = st

            quarter = jnp.minimum(g_ // (QSEG // 16), NFLUSH - 1)

            @pl.when(quarter > flushed)
            def _():
                pltpu.async_copy(
                    out_v, res_hbm.at[pl.ds(seg0 + flushed * QSEG, QSEG)],
                    semf,
                ).wait()

            flushed_n = jnp.maximum(flushed, quarter)

            ev = es_v[pl.ds(g_ * 16, 16)]
            in_mask = (ev >= eb) & (ev < hi)
            el = jnp.where(in_mask, ev - eb, 0)
            rows = jnp.where(
                in_mask, (g_ % (QSEG // 16)) * 16 + lanes, 0
            )
            pv = jnp.full((16,), p, dtype=jnp.int32)
            for h in range(D):
                fr = jnp.full((16,), h // 8, dtype=jnp.int32)
                fc = jnp.full((16,), h % 8, dtype=jnp.int32)
                hv = jnp.full((16,), h, dtype=jnp.int32)
                v = plsc.load_gather(slab_v, [pv, fr, fc, el])
                v = 1.0 / (1.0 + jnp.exp(-v))
                plsc.store_scatter(out_v, [rows, hv], v, mask=in_mask)
            adv = jnp.max(ev) < hi
            g_n = jnp.where(adv, g_ + 1, g_)
            done_n = jnp.logical_not(adv) | (g_n >= SEG // 16)
            return (g_n, flushed_n, done_n)

        g, flushed, _ = lax.while_loop(
            cond, body, (carry[0], carry[1], jnp.bool_(False))
        )
        return (g, flushed)

    start_chunk(0, 0)

    def pair_body(j, carry):
        k0 = 2 * j
        k1 = 2 * j + 1

        @pl.when(k1 < cnt)
        def _():
            start_chunk(k1, 1)

        def do0(st):
            wait_chunk(k0, 0)
            return process_chunk(k0, 0, st)

        carry = lax.cond(k0 < cnt, do0, lambda st: st, carry)

        @pl.when(k1 + 1 < cnt)
        def _():
            start_chunk(k1 + 1, 0)

        def do1(st):
            wait_chunk(k1, 1)
            return process_chunk(k1, 1, st)

        carry = lax.cond(k1 < cnt, do1, lambda st: st, carry)
        return carry

    lax.fori_loop(
        0, (cnt + 1) // 2, pair_body, (jnp.int32(0), jnp.int32(0))
    )

    pltpu.async_copy(
        out_v, res_hbm.at[pl.ds(seg0 + (NFLUSH - 1) * QSEG, QSEG)], semf
    ).wait()


@functools.partial(
    pl.kernel,
    mesh=_mesh,
    out_type=jax.ShapeDtypeStruct((B, ROW_PAD), jnp.float32),
    scratch_types=[
        pltpu.VMEM((SEG // 128, 128), jnp.int32),
        pltpu.VMEM((SEG, ROW_PAD), jnp.float32),
        pltpu.SemaphoreType.DMA,
    ],
    compiler_params=_params,
)
def _unsort(inv_hbm, res_hbm, out_hbm, inv_v, rows_v, sem):
    wid = lax.axis_index("s") * NC + lax.axis_index("c")
    base = wid * SEG
    pltpu.sync_copy(inv_hbm.at[pl.ds(wid * (SEG // 128), SEG // 128)], inv_v)
    copies = []
    for g in range(SEG // 128):
        copies.append(
            pltpu.async_copy(
                res_hbm.at[inv_v.at[g]],
                rows_v.at[pl.ds(g * 128, 128)],
                sem,
            )
        )
    for c in copies:
        c.wait()
    pltpu.sync_copy(rows_v, out_hbm.at[pl.ds(base, SEG)])


def kernel(X, Y, gate_theta):
    del X
    y32 = Y.astype(jnp.int32)
    iota = lax.broadcasted_iota(jnp.int32, (B,), 0)
    es, order = lax.sort([y32, iota], num_keys=1)
    inv = jnp.zeros((B,), jnp.int32).at[order].set(iota)
    tbl = gate_theta.T.reshape(8, 8, NUM_E)
    tail = jnp.pad(
        tbl[:, :, TAIL_START:],
        ((0, 0), (0, 0), (0, TAIL_PAD - (NUM_E - TAIL_START))),
    )
    res = _gather_sigmoid(es, tbl, tail)
    out = _unsort(inv.reshape(B // 128, 128), res)
    return out[:, :D]
